# baseline (speedup 1.0000x reference)
import jax
import jax.numpy as jnp
from jax.experimental import pallas as pl
from jax.experimental.pallas import tpu as pltpu

_IN_C = 192
_OUT_C = 384
_W = 0.5


_NSLOT = 4


def _body(dest_ref, x_ref, out_ref, scratch, sem):
    i = pl.program_id(0)
    slot = jax.lax.rem(i, _NSLOT)

    def _copies(step, s):
        d0 = dest_ref[step]
        d1 = dest_ref[_IN_C + step]
        return [
            pltpu.make_async_copy(
                scratch.at[s], out_ref.at[:, pl.ds(d, 1)], sem.at[s, k]
            )
            for k, d in enumerate((d0, d1))
        ]

    @pl.when(i >= _NSLOT)
    def _():
        for c in _copies(i - _NSLOT, slot):
            c.wait()

    scratch[slot] = x_ref[...] * _W

    for c in _copies(i, slot):
        c.start()

    @pl.when(i == _IN_C - 1)
    def _():
        for back in range(_NSLOT - 1, -1, -1):
            for c in _copies(i - back, jax.lax.rem(i - back, _NSLOT)):
                c.wait()


def kernel(x, perm):
    B, C, H, W = x.shape
    HW = H * W
    S = HW // 128
    xr = x.reshape(B, C, S, 128)

    perm32 = perm.astype(jnp.int32)
    ar = jnp.arange(_IN_C, dtype=jnp.int32)
    z = jnp.zeros((_IN_C,), jnp.int32)
    dest0 = z.at[perm32[:_IN_C]].set(ar)
    dest1 = z.at[perm32[_IN_C:]].set(ar) + _IN_C
    dests = jnp.concatenate([dest0, dest1])

    out = pl.pallas_call(
        _body,
        grid_spec=pltpu.PrefetchScalarGridSpec(
            num_scalar_prefetch=1,
            grid=(_IN_C,),
            in_specs=[
                pl.BlockSpec((B, 1, S, 128), lambda i, dest_ref: (0, i, 0, 0))
            ],
            out_specs=pl.BlockSpec(memory_space=pl.MemorySpace.ANY),
            scratch_shapes=[
                pltpu.VMEM((_NSLOT, B, 1, S, 128), jnp.float32),
                pltpu.SemaphoreType.DMA((_NSLOT, 2)),
            ],
        ),
        out_shape=jax.ShapeDtypeStruct((B, _OUT_C, S, 128), x.dtype),
    )(dests, xr)
    return out.reshape(B, _OUT_C, H, W)
